# Initial kernel scaffold; baseline (speedup 1.0000x reference)
import jax
import jax.numpy as jnp
from jax.experimental import pallas as pl


def kernel(x, edge_index, rbf_feature, angle_feature, W_edge, b_edge, W1, b1, W2, b2):
    raise NotImplementedError("write your pallas kernel here")


# speedup vs baseline: 1.7172x; 1.7172x over previous
import functools

import jax
import jax.numpy as jnp
from jax import lax
from jax.experimental import pallas as pl
from jax.experimental.pallas import tpu as pltpu
from jax.experimental.pallas import tpu_sc as plsc

N_NODES = 10000
N_EDGES = 320000
EMBED = 128
RBF = 64
ANG = 32

NC = 2
NS = 16
EC = 128
ROWS = N_EDGES // EC
N_PAD = 10240
NODES_PER_SUB = N_PAD // NS
FULL_ITERS = ROWS // NS
EXTRA_SUBS = ROWS - NS * FULL_ITERS


def _sc_body(row_hbm, rbf_hbm, ang_hbm, out_hbm, idx_v, buf_v, stage_v, acc):
    c = lax.axis_index("c")
    s = lax.axis_index("s")

    z16 = jnp.zeros((16,), jnp.float32)

    def _zero_row(i, _):
        for k in range(RBF // 16):
            stage_v[i, pl.ds(k * 16, 16)] = z16
        return 0

    lax.fori_loop(0, NODES_PER_SUB, _zero_row, 0)
    n0 = s * NODES_PER_SUB
    pltpu.sync_copy(stage_v, acc.at[pl.ds(n0, NODES_PER_SUB)])

    @pl.when(c == 1)
    def _():
        one16 = jnp.ones((16,), jnp.float32)

        def _fill_ones(i, _):
            buf_v[i, pl.ds(ANG, 16)] = one16
            buf_v[i, pl.ds(ANG + 16, 16)] = one16
            return 0

        lax.fori_loop(0, EC, _fill_ones, 0)

    plsc.subcore_barrier()

    n_iter = jnp.where(s < EXTRA_SUBS, FULL_ITERS + 1, FULL_ITERS)

    def _edge_step(i, _):
        r = s + NS * i
        pltpu.sync_copy(row_hbm.at[r], idx_v)
        e0 = r * EC

        @pl.when(c == 0)
        def _():
            pltpu.sync_copy(rbf_hbm.at[pl.ds(e0, EC)], buf_v)
            pltpu.sync_copy(buf_v, acc.at[idx_v.at[0]], add=True)

        @pl.when(c == 1)
        def _():
            pltpu.sync_copy(ang_hbm.at[pl.ds(e0, EC)],
                            buf_v.at[:, pl.ds(0, ANG)])
            pltpu.sync_copy(buf_v, acc.at[idx_v.at[0]], add=True)

        return 0

    lax.fori_loop(0, n_iter, _edge_step, 0)
    plsc.subcore_barrier()

    pltpu.sync_copy(acc.at[pl.ds(n0, NODES_PER_SUB)], stage_v)
    pltpu.sync_copy(stage_v, out_hbm.at[pl.ds(c * N_PAD + n0, NODES_PER_SUB)])


_sc_scatter = functools.partial(
    pl.kernel,
    out_type=jax.ShapeDtypeStruct((NC * N_PAD, RBF), jnp.float32),
    mesh=plsc.VectorSubcoreMesh(core_axis_name="c", subcore_axis_name="s",
                                num_cores=NC, num_subcores=NS),
    scratch_types=[
        pltpu.VMEM((1, EC), jnp.int32),
        pltpu.VMEM((EC, RBF), jnp.float32),
        pltpu.VMEM((NODES_PER_SUB, RBF), jnp.float32),
        pltpu.VMEM_SHARED((N_PAD, RBF), jnp.float32),
    ],
    compiler_params=pltpu.CompilerParams(use_tc_tiling_on_sc=False),
)(_sc_body)


BLK = 2000


def _tc_body(pr, x, we, be, w1, b1, w2, b2, o):
    aggr = pr[0]
    agga = pr[1, :, 0:ANG]
    cnt = pr[1, :, ANG:ANG + 1]
    hi = jax.lax.Precision.HIGHEST
    agg = (jnp.dot(aggr, we[:RBF], precision=hi,
                   preferred_element_type=jnp.float32)
           + jnp.dot(agga, we[RBF:], precision=hi,
                     preferred_element_type=jnp.float32)
           + cnt * be[...])
    h = jnp.maximum(jnp.dot(agg, w1[...], precision=hi,
                            preferred_element_type=jnp.float32) + b1[...], 0.0)
    h = jnp.dot(h, w2[...], precision=hi,
                preferred_element_type=jnp.float32) + b2[...]
    o[...] = x[...] + h


def _tc_combine(pr, x, W_edge, b_edge, W1, b1, W2, b2):
    grid = N_NODES // BLK
    return pl.pallas_call(
        _tc_body,
        grid=(grid,),
        in_specs=[
            pl.BlockSpec((NC, BLK, RBF), lambda i: (0, i, 0)),
            pl.BlockSpec((BLK, EMBED), lambda i: (i, 0)),
            pl.BlockSpec((RBF + ANG, EMBED), lambda i: (0, 0)),
            pl.BlockSpec((1, EMBED), lambda i: (0, 0)),
            pl.BlockSpec((EMBED, EMBED), lambda i: (0, 0)),
            pl.BlockSpec((1, EMBED), lambda i: (0, 0)),
            pl.BlockSpec((EMBED, EMBED), lambda i: (0, 0)),
            pl.BlockSpec((1, EMBED), lambda i: (0, 0)),
        ],
        out_specs=pl.BlockSpec((BLK, EMBED), lambda i: (i, 0)),
        out_shape=jax.ShapeDtypeStruct((N_NODES, EMBED), jnp.float32),
    )(pr, x, W_edge, b_edge, W1, b1, W2, b2)


def kernel(x, edge_index, rbf_feature, angle_feature, W_edge, b_edge,
           W1, b1, W2, b2):
    row = edge_index[0].astype(jnp.int32).reshape(ROWS, 1, EC)
    partials = _sc_scatter(row, rbf_feature, angle_feature)
    pr = partials.reshape(NC, N_PAD, RBF)
    return _tc_combine(pr, x,
                       W_edge, b_edge.reshape(1, EMBED),
                       W1, b1.reshape(1, EMBED),
                       W2, b2.reshape(1, EMBED))
